# baseline (speedup 1.0000x reference)
import jax
import jax.numpy as jnp
from jax import lax
from jax.experimental import pallas as pl
from jax.experimental.pallas import tpu as pltpu
from jax.experimental.pallas import tpu_sc as plsc

_N = 16
_M = 64
_B = 1048576

_NW = 32
_BPW = _B // _NW
_CH = 2048
_NCHUNK = _BPW // _CH
_NBUF = 3
_L = 16
_RSTRIDE = _M * _N + 1


def _logsm_body(mt_ref, out_ref):
    x = mt_ref[...]
    mx = jnp.max(x, axis=0, keepdims=True)
    s = x - mx
    lse = jnp.log(jnp.sum(jnp.exp(s), axis=0, keepdims=True))
    out_ref[...] = s - lse


def _make_table(matrix):
    return pl.pallas_call(
        _logsm_body,
        out_shape=jax.ShapeDtypeStruct((_M, _N), jnp.float32),
    )(matrix.T)


def _gather_body(trep_hbm, xt_hbm, out_hbm, trep_v, idx_v, rows_v, *sems):
    si = sems[0:_NBUF]
    sw = sems[_NBUF:2 * _NBUF]
    wid = lax.axis_index("s") * 2 + lax.axis_index("c")
    base = wid * _BPW

    pltpu.sync_copy(trep_hbm, trep_v)

    def start_idx(b, i):
        return pltpu.async_copy(
            xt_hbm.at[pl.ds(base + i * _CH, _CH)], idx_v.at[b], si[b])

    def start_write(b, i):
        return pltpu.async_copy(
            rows_v.at[b], out_hbm.at[:, pl.ds(base + i * _CH, _CH)], sw[b])

    lane_base = lax.iota(jnp.int32, _L) * _RSTRIDE
    cvec = [lane_base + n for n in range(_N)]

    def compute_chunk(b):
        def group(g, carry):
            o = g * _L
            tv = idx_v[b, pl.ds(o, _L)] * _N
            for n in range(_N):
                vals = plsc.load_gather(trep_v, [tv + cvec[n]])
                rows_v[b, n, pl.ds(o, _L)] = vals
            return carry

        lax.fori_loop(0, _CH // _L, group, 0)

    h_idx = [None] * _NBUF
    h_w = [None] * _NBUF
    for i in range(_NBUF):
        h_idx[i] = start_idx(i, i)
    for i in range(_NCHUNK):
        b = i % _NBUF
        h_idx[b].wait()
        if i >= _NBUF:
            h_w[b].wait()
        compute_chunk(b)
        h_w[b] = start_write(b, i)
        if i + _NBUF < _NCHUNK:
            h_idx[b] = start_idx(b, i + _NBUF)
    for b in range(_NBUF):
        h_w[b].wait()


def kernel(matrix, x_t):
    table = _make_table(matrix)
    flat = table.reshape(-1)
    trep = jnp.pad(jnp.broadcast_to(flat, (_L, _M * _N)),
                   ((0, 0), (0, 1))).reshape(-1)
    f = pl.kernel(
        _gather_body,
        out_type=jax.ShapeDtypeStruct((_N, _B), jnp.float32),
        mesh=plsc.VectorSubcoreMesh(core_axis_name="c", subcore_axis_name="s"),
        scratch_types=[
            pltpu.VMEM((_L * _RSTRIDE,), jnp.float32),
            pltpu.VMEM((_NBUF, _CH), jnp.int32),
            pltpu.VMEM((_NBUF, _N, _CH), jnp.float32),
        ] + [pltpu.SemaphoreType.DMA] * (2 * _NBUF),
        compiler_params=pltpu.CompilerParams(
            use_tc_tiling_on_sc=False, needs_layout_passes=False),
    )
    return f(trep, x_t).T


# speedup vs baseline: 9.2720x; 9.2720x over previous
import jax
import jax.numpy as jnp
from jax import lax
from jax.experimental import pallas as pl
from jax.experimental.pallas import tpu as pltpu
from jax.experimental.pallas import tpu_sc as plsc

_N = 16
_M = 64
_B = 1048576

_NW = 32
_BPW = _B // _NW
_CH = 2048
_NCHUNK = _BPW // _CH
_NBUF = 3
_L = 16
_RSTRIDE = _M * _N + 1


def _logsm_body(mt_ref, out_ref):
    x = mt_ref[...]
    mx = jnp.max(x, axis=0, keepdims=True)
    s = x - mx
    lse = jnp.log(jnp.sum(jnp.exp(s), axis=0, keepdims=True))
    out_ref[...] = s - lse


def _make_table(matrix):
    return pl.pallas_call(
        _logsm_body,
        out_shape=jax.ShapeDtypeStruct((_M, _N), jnp.float32),
    )(matrix.T)


def _gather_body(trep_hbm, xt_hbm, out_hbm, trep_v, idx_v, rows_v, *sems):
    si = sems[0:_NBUF]
    sw = sems[_NBUF:2 * _NBUF]
    wid = lax.axis_index("s") * 2 + lax.axis_index("c")
    base = wid * _BPW

    pltpu.sync_copy(trep_hbm, trep_v)

    def start_idx(b, i):
        return pltpu.async_copy(
            xt_hbm.at[pl.ds(base + i * _CH, _CH)], idx_v.at[b], si[b])

    def start_write(b, i):
        off = (base + i * _CH) * 8
        h0 = pltpu.async_copy(
            rows_v.at[b, 0], out_hbm.at[0, pl.ds(off, _CH * 8)], sw[b])
        h1 = pltpu.async_copy(
            rows_v.at[b, 1], out_hbm.at[1, pl.ds(off, _CH * 8)], sw[b])
        return (h0, h1)

    lane_base = lax.iota(jnp.int32, _L) * _RSTRIDE
    cvec = [lane_base + n for n in range(_N)]

    def compute_chunk(b):
        def group(g, carry):
            o = g * _L
            tv = idx_v[b, pl.ds(o, _L)] * _N
            boff = (g // 8) * 1024 + (g % 8) * _L
            for n in range(_N):
                vals = plsc.load_gather(trep_v, [tv + cvec[n]])
                rows_v[b, n // 8, pl.ds(boff + (n % 8) * 128, _L)] = vals
            return carry

        lax.fori_loop(0, _CH // _L, group, 0)

    h_idx = [None] * _NBUF
    h_w = [None] * _NBUF
    for i in range(_NBUF):
        h_idx[i] = start_idx(i, i)
    for i in range(_NCHUNK):
        b = i % _NBUF
        h_idx[b].wait()
        if i >= _NBUF:
            for h in h_w[b]:
                h.wait()
        compute_chunk(b)
        h_w[b] = start_write(b, i)
        if i + _NBUF < _NCHUNK:
            h_idx[b] = start_idx(b, i + _NBUF)
    for b in range(_NBUF):
        for h in h_w[b]:
            h.wait()


def kernel(matrix, x_t):
    table = _make_table(matrix)
    flat = table.reshape(-1)
    trep = jnp.pad(jnp.broadcast_to(flat, (_L, _M * _N)),
                   ((0, 0), (0, 1))).reshape(-1)
    f = pl.kernel(
        _gather_body,
        out_type=jax.ShapeDtypeStruct((2, _B * 8), jnp.float32),
        mesh=plsc.VectorSubcoreMesh(core_axis_name="c", subcore_axis_name="s"),
        scratch_types=[
            pltpu.VMEM((_L * _RSTRIDE,), jnp.float32),
            pltpu.VMEM((_NBUF, _CH), jnp.int32),
            pltpu.VMEM((_NBUF, 2, _CH * 8), jnp.float32),
        ] + [pltpu.SemaphoreType.DMA] * (2 * _NBUF),
        compiler_params=pltpu.CompilerParams(
            use_tc_tiling_on_sc=False, needs_layout_passes=False),
    )
    out4 = f(trep, x_t).reshape(2, _B // 128, 8, 128)
    return out4.transpose(1, 3, 0, 2).reshape(_B, _N)
